# Initial kernel scaffold; baseline (speedup 1.0000x reference)
import functools

import jax
import jax.numpy as jnp
from jax import lax
from jax.experimental import pallas as pl
from jax.experimental.pallas import tpu as pltpu
from jax.experimental.pallas import tpu_sc as plsc

_B = 4096
_F = 26
_CARD = 4000
_D = 64
_TOTAL = _B * _F
_NW = 32
_R = _TOTAL // _NW
_CHUNK = 128
_NCHUNK = _R // _CHUNK
_GROUP = 13
_NGROUP = _NCHUNK // _GROUP
_GROWS = _GROUP * _CHUNK


def _make_gather():
    mesh = plsc.VectorSubcoreMesh(core_axis_name="c", subcore_axis_name="s")

    @functools.partial(
        pl.kernel,
        mesh=mesh,
        out_type=jax.ShapeDtypeStruct((_TOTAL, _D), jnp.float32),
        scratch_types=[
            pltpu.VMEM((_R,), jnp.int32),
            pltpu.VMEM((_GROWS, _D), jnp.float32),
            pltpu.SemaphoreType.DMA,
        ],
    )
    def gather_kernel(idx_hbm, table_hbm, out_hbm, idx_v, rows_v, sem):
        wid = lax.axis_index("s") * 2 + lax.axis_index("c")
        base = wid * _R

        pltpu.sync_copy(idx_hbm.at[pl.ds(base, _R)], idx_v)

        def adjust(j, carry):
            g = j * 16 + lax.iota(jnp.int32, 16)
            off = (g % _F) * _CARD
            idx_v[pl.ds(j * 16, 16)] = idx_v[pl.ds(j * 16, 16)] + off
            return carry

        lax.fori_loop(0, _R // 16, adjust, 0)

        for grp in range(_NGROUP):
            copies = []
            for k in range(_GROUP):
                c = grp * _GROUP + k
                copies.append(
                    pltpu.async_copy(
                        table_hbm.at[idx_v.at[pl.ds(c * _CHUNK, _CHUNK)]],
                        rows_v.at[pl.ds(k * _CHUNK, _CHUNK)],
                        sem,
                    )
                )
            for cp in copies:
                cp.wait()
            pltpu.sync_copy(
                rows_v, out_hbm.at[pl.ds(base + grp * _GROWS, _GROWS)]
            )

    return gather_kernel


_gather = _make_gather()


def kernel(inputs, table):
    idx_flat = inputs.reshape(_TOTAL).astype(jnp.int32)
    out = _gather(idx_flat, table)
    return out.reshape(_B, _F, _D)


# speedup vs baseline: 1.1957x; 1.1957x over previous
import functools

import jax
import jax.numpy as jnp
from jax import lax
from jax.experimental import pallas as pl
from jax.experimental.pallas import tpu as pltpu
from jax.experimental.pallas import tpu_sc as plsc

_B = 4096
_F = 26
_CARD = 4000
_D = 64
_TOTAL = _B * _F
_NW = 32
_R = _TOTAL // _NW
_CHUNK = 128
_NCHUNK = _R // _CHUNK
_GROUP = 13
_NGROUP = _NCHUNK // _GROUP
_GROWS = _GROUP * _CHUNK


@functools.lru_cache(maxsize=None)
def _make_gather():
    mesh = plsc.VectorSubcoreMesh(core_axis_name="c", subcore_axis_name="s")

    @functools.partial(
        pl.kernel,
        mesh=mesh,
        out_type=jax.ShapeDtypeStruct((_TOTAL, _D), jnp.float32),
        compiler_params=pltpu.CompilerParams(use_tc_tiling_on_sc=False),
        scratch_types=[
            pltpu.VMEM((_R,), jnp.int32),
            pltpu.VMEM((_GROWS, _D), jnp.float32),
            pltpu.SemaphoreType.DMA,
        ],
    )
    def gather_kernel(idx_hbm, table_hbm, out_hbm, idx_v, rows_v, sem):
        wid = lax.axis_index("s") * 2 + lax.axis_index("c")
        base = wid * _R

        pltpu.sync_copy(idx_hbm.at[pl.ds(base, _R)], idx_v)

        def adjust(j, carry):
            g = j * 16 + lax.iota(jnp.int32, 16)
            off = (g % _F) * _CARD
            idx_v[pl.ds(j * 16, 16)] = idx_v[pl.ds(j * 16, 16)] + off
            return carry

        lax.fori_loop(0, _R // 16, adjust, 0)

        for grp in range(_NGROUP):
            copies = []
            for k in range(_GROUP):
                c = grp * _GROUP + k
                copies.append(
                    pltpu.async_copy(
                        table_hbm.at[idx_v.at[pl.ds(c * _CHUNK, _CHUNK)]],
                        rows_v.at[pl.ds(k * _CHUNK, _CHUNK)],
                        sem,
                    )
                )
            for cp in copies:
                cp.wait()
            pltpu.sync_copy(
                rows_v, out_hbm.at[pl.ds(base + grp * _GROWS, _GROWS)]
            )

    return gather_kernel


def kernel(inputs, table):
    idx_flat = inputs.reshape(_TOTAL).astype(jnp.int32)
    out = _make_gather()(idx_flat, table)
    return out.reshape(_B, _F, _D)
